# baseline (speedup 1.0000x reference)
import functools

import jax
import jax.numpy as jnp
from jax import lax
from jax.experimental import pallas as pl
from jax.experimental.pallas import tpu as pltpu
from jax.experimental.pallas import tpu_sc as plsc

_NORM = 11.0
_INV_NORM = 1.0 / 11.0
_LN2 = 0.6931471805599453
_LC = (0.9999987635044434, -0.4998719159347707, 0.3311205190977852,
       -0.23514863754145301, 0.14943458362585965, -0.06658804993607115,
       0.014202825621276338)

_W = 512
_STRIP = 8 * _W
_NW = 32
_TOTAL = 2 * 1 * 16 * 512 * 512
_N_STRIPS = _TOTAL // _STRIP
_STRIPS_PER_W = _N_STRIPS // _NW
_BLOCKS_PER_STRIP = _W // 8


def _merge16(a, b):
    rb = jnp.flip(b)
    lo = jnp.minimum(a, rb)
    hi = jnp.maximum(a, rb)
    return jnp.sort(lo), jnp.sort(hi)


def _sort64(v0, v1, v2, v3):
    s0, s1, s2, s3 = jnp.sort(v0), jnp.sort(v1), jnp.sort(v2), jnp.sort(v3)
    a0, a1 = _merge16(s0, s1)
    b0, b1 = _merge16(s2, s3)
    rb0, rb1 = jnp.flip(b1), jnp.flip(b0)
    l0 = jnp.minimum(a0, rb0)
    l1 = jnp.minimum(a1, rb1)
    h0 = jnp.maximum(a0, rb0)
    h1 = jnp.maximum(a1, rb1)
    m0 = jnp.minimum(l0, l1)
    m1 = jnp.maximum(l0, l1)
    m2 = jnp.minimum(h0, h1)
    m3 = jnp.maximum(h0, h1)
    return jnp.sort(m0), jnp.sort(m1), jnp.sort(m2), jnp.sort(m3)


def _log1p_over_norm(q):
    y = q + 1.0
    b = lax.bitcast_convert_type(y, jnp.int32)
    e = (b >> 23) - 127
    m = lax.bitcast_convert_type((b & 0x7FFFFF) | 0x3F800000, jnp.float32)
    ef = e.astype(jnp.float32)
    z = (m - 1.0) / (m + 1.0)
    z2 = z * z
    p = (1.0 / 7.0 * z2 + 0.2) * z2 + 1.0 / 3.0
    p = p * z2 + 1.0
    return (ef * _LN2 + 2.0 * z * p) * _INV_NORM


def _expm1_norm(v):
    return jnp.exp(v * _NORM) - 1.0


def _bcast_lane(x, idx):
    return lax.gather(
        x, idx[:, None],
        dimension_numbers=lax.GatherDimensionNumbers(
            offset_dims=(), collapsed_slice_dims=(0,), start_index_map=(0,)),
        slice_sizes=(1,),
        mode=lax.GatherScatterMode.PROMISE_IN_BOUNDS)


@functools.partial(
    pl.kernel,
    out_type=jax.ShapeDtypeStruct((_TOTAL,), jnp.float32),
    mesh=plsc.VectorSubcoreMesh(core_axis_name="c", subcore_axis_name="s"),
    scratch_types=[
        pltpu.VMEM((_STRIP,), jnp.float32),
        pltpu.VMEM((_STRIP,), jnp.float32),
        pltpu.VMEM((_STRIP,), jnp.float32),
        pltpu.VMEM((_STRIP,), jnp.float32),
        pltpu.VMEM((_STRIP,), jnp.float32),
        pltpu.VMEM((_STRIP,), jnp.float32),
        pltpu.SemaphoreType.DMA,
        pltpu.SemaphoreType.DMA,
        pltpu.SemaphoreType.DMA,
        pltpu.SemaphoreType.DMA,
        pltpu.SemaphoreType.DMA,
        pltpu.SemaphoreType.DMA,
        pltpu.VMEM((_STRIP,), jnp.float32),
        pltpu.VMEM((16 * _BLOCKS_PER_STRIP,), jnp.float32),
    ],
    compiler_params=pltpu.CompilerParams(needs_layout_passes=False),
)
def _wf_kernel(pred_hbm, inp_hbm, out_hbm,
               pred_v0, inp_v0, out_v0, pred_v1, inp_v1, out_v1,
               sem_p0, sem_i0, sem_o0, sem_p1, sem_i1, sem_o1,
               p_buf, th_buf):
    nc = 2
    wid = lax.axis_index("s") * nc + lax.axis_index("c")
    base_off = wid * _STRIPS_PER_W * _STRIP
    io = lax.iota(jnp.int32, 16)
    row = io >> 3
    col = io & 7
    base = [(2 * k + row) * 128 + col for k in range(4)]
    kf = (io + 1).astype(jnp.float32)
    inv_k = [1.0 / (kf + (16.0 * j)) for j in range(4)]
    lane15 = jnp.full((16,), 15, jnp.int32)

    slots = (
        (pred_v0, inp_v0, out_v0, sem_p0, sem_i0, sem_o0),
        (pred_v1, inp_v1, out_v1, sem_p1, sem_i1, sem_o1),
    )

    def start_in(s, slot):
        pred_v, inp_v, _, sem_p, sem_i, _ = slot
        off = base_off + s * _STRIP
        pltpu.async_copy(pred_hbm.at[pl.ds(off, _STRIP)], pred_v, sem_p)
        pltpu.async_copy(inp_hbm.at[pl.ds(off, _STRIP)], inp_v, sem_i)

    def wait_in(s, slot):
        pred_v, inp_v, _, sem_p, sem_i, _ = slot
        off = base_off + s * _STRIP
        pltpu.make_async_copy(pred_hbm.at[pl.ds(off, _STRIP)], pred_v, sem_p).wait()
        pltpu.make_async_copy(inp_hbm.at[pl.ds(off, _STRIP)], inp_v, sem_i).wait()

    def wait_out(s, slot):
        _, _, out_v, _, _, sem_o = slot
        off = base_off + s * _STRIP
        pltpu.make_async_copy(out_v, out_hbm.at[pl.ds(off, _STRIP)], sem_o).wait()

    def compute(slot):
        pred_v, inp_v, out_v = slot[0], slot[1], slot[2]

        @plsc.parallel_loop(0, _BLOCKS_PER_STRIP, 1, unroll=4)
        def blk(bi):
            off_b = ((bi >> 4) << 10) + ((bi & 15) << 3)
            idx = [base[k] + off_b for k in range(4)]
            pv = [_expm1_norm(plsc.load_gather(pred_v, [idx[k]])) for k in range(4)]
            for k in range(4):
                p_buf[pl.ds(bi * 64 + k * 16, 16)] = pv[k]
            iv = [_expm1_norm(plsc.load_gather(inp_v, [idx[k]])) for k in range(4)]
            S = _bcast_lane(plsc.cumsum((iv[0] + iv[1]) + (iv[2] + iv[3])),
                            lane15)
            m0, m1, m2, m3 = _sort64(pv[0], pv[1], pv[2], pv[3])
            u = [jnp.flip(m3), jnp.flip(m2), jnp.flip(m1), jnp.flip(m0)]
            acc_m = jnp.zeros((16,), jnp.float32)
            rho_i = jnp.zeros((16,), jnp.int32)
            carry_v = jnp.zeros((16,), jnp.float32)
            for j in range(4):
                cj = plsc.cumsum(u[j]) + carry_v
                carry_v = _bcast_lane(cj, lane15)
                t = (cj - S) * inv_k[j]
                mask = u[j] > t
                acc_m = acc_m + jnp.where(mask, u[j], 0.0)
                rho_i = rho_i + plsc.all_reduce_population_count(mask)
            msum = _bcast_lane(plsc.cumsum(acc_m), lane15)
            rho = rho_i.astype(jnp.float32)
            maxp = _bcast_lane(m3, lane15)
            theta = jnp.where(rho > 0.5, (msum - S) / jnp.maximum(rho, 1.0),
                              maxp - S)
            th_buf[pl.ds(bi * 16, 16)] = theta

        @plsc.parallel_loop(0, _BLOCKS_PER_STRIP, 1, unroll=4)
        def blk_out(bi):
            off_b = ((bi >> 4) << 10) + ((bi & 15) << 3)
            theta = th_buf[pl.ds(bi * 16, 16)]
            for k in range(4):
                pv = p_buf[pl.ds(bi * 64 + k * 16, 16)]
                q = jnp.maximum(pv - theta, 0.0)
                plsc.store_scatter(out_v, [base[k] + off_b], _log1p_over_norm(q))

    def start_out(s, slot):
        _, _, out_v, _, _, sem_o = slot
        off = base_off + s * _STRIP
        pltpu.async_copy(out_v, out_hbm.at[pl.ds(off, _STRIP)], sem_o)

    start_in(0, slots[0])

    def pair_body(ii, carry):
        s0 = ii * 2
        start_in(s0 + 1, slots[1])
        wait_in(s0, slots[0])

        @pl.when(ii > 0)
        def _():
            wait_out(s0 - 2, slots[0])

        compute(slots[0])
        start_out(s0, slots[0])

        @pl.when(ii < _STRIPS_PER_W // 2 - 1)
        def _():
            start_in(s0 + 2, slots[0])

        wait_in(s0 + 1, slots[1])

        @pl.when(ii > 0)
        def _():
            wait_out(s0 - 1, slots[1])

        compute(slots[1])
        start_out(s0 + 1, slots[1])
        return carry

    lax.fori_loop(0, _STRIPS_PER_W // 2, pair_body, 0, unroll=False)
    wait_out(_STRIPS_PER_W - 2, slots[0])
    wait_out(_STRIPS_PER_W - 1, slots[1])


def _to_tile_order(x):
    return (x.reshape(32, 64, 8, 4, 128)
             .transpose(0, 1, 3, 2, 4)
             .reshape(-1))


def kernel(pred_log_norm, input_mosaic_log_norm):
    shape = pred_log_norm.shape
    pf = _to_tile_order(pred_log_norm)
    nf = _to_tile_order(input_mosaic_log_norm)
    out = _wf_kernel(pf, nf)
    return (out.reshape(32, 64, 4, 8, 128)
               .transpose(0, 1, 3, 2, 4)
               .reshape(shape))


# speedup vs baseline: 1.1300x; 1.1300x over previous
import functools

import jax
import jax.numpy as jnp
from jax import lax
from jax.experimental import pallas as pl
from jax.experimental.pallas import tpu as pltpu
from jax.experimental.pallas import tpu_sc as plsc

_NORM = 11.0
_INV_NORM = 1.0 / 11.0
_LN2 = 0.6931471805599453
_LC = (0.9999987635044434, -0.4998719159347707, 0.3311205190977852,
       -0.23514863754145301, 0.14943458362585965, -0.06658804993607115,
       0.014202825621276338)

_W = 512
_STRIP = 8 * _W
_NW = 32
_TOTAL = 2 * 1 * 16 * 512 * 512
_N_STRIPS = _TOTAL // _STRIP
_STRIPS_PER_W = _N_STRIPS // _NW
_BLOCKS_PER_STRIP = _W // 8


def _merge16(a, b):
    rb = jnp.flip(b)
    lo = jnp.minimum(a, rb)
    hi = jnp.maximum(a, rb)
    return jnp.sort(lo), jnp.sort(hi)


def _sort64(v0, v1, v2, v3):
    s0, s1, s2, s3 = jnp.sort(v0), jnp.sort(v1), jnp.sort(v2), jnp.sort(v3)
    a0, a1 = _merge16(s0, s1)
    b0, b1 = _merge16(s2, s3)
    rb0, rb1 = jnp.flip(b1), jnp.flip(b0)
    l0 = jnp.minimum(a0, rb0)
    l1 = jnp.minimum(a1, rb1)
    h0 = jnp.maximum(a0, rb0)
    h1 = jnp.maximum(a1, rb1)
    m0 = jnp.minimum(l0, l1)
    m1 = jnp.maximum(l0, l1)
    m2 = jnp.minimum(h0, h1)
    m3 = jnp.maximum(h0, h1)
    return jnp.sort(m0), jnp.sort(m1), jnp.sort(m2), jnp.sort(m3)


def _log1p_over_norm(q):
    y = q + 1.0
    b = lax.bitcast_convert_type(y, jnp.int32)
    e = (b >> 23) - 127
    m = lax.bitcast_convert_type((b & 0x7FFFFF) | 0x3F800000, jnp.float32)
    ef = e.astype(jnp.float32)
    z = (m - 1.0) / (m + 1.0)
    z2 = z * z
    p = (1.0 / 7.0 * z2 + 0.2) * z2 + 1.0 / 3.0
    p = p * z2 + 1.0
    return (ef * _LN2 + 2.0 * z * p) * _INV_NORM


def _expm1_norm(v):
    return jnp.exp(v * _NORM) - 1.0


def _bcast_lane(x, idx):
    return lax.gather(
        x, idx[:, None],
        dimension_numbers=lax.GatherDimensionNumbers(
            offset_dims=(), collapsed_slice_dims=(0,), start_index_map=(0,)),
        slice_sizes=(1,),
        mode=lax.GatherScatterMode.PROMISE_IN_BOUNDS)


@functools.partial(
    pl.kernel,
    out_type=jax.ShapeDtypeStruct((_TOTAL,), jnp.float32),
    mesh=plsc.VectorSubcoreMesh(core_axis_name="c", subcore_axis_name="s"),
    scratch_types=[
        pltpu.VMEM((_STRIP,), jnp.float32),
        pltpu.VMEM((_STRIP,), jnp.float32),
        pltpu.VMEM((_STRIP,), jnp.float32),
        pltpu.VMEM((_STRIP,), jnp.float32),
        pltpu.VMEM((_STRIP,), jnp.float32),
        pltpu.VMEM((_STRIP,), jnp.float32),
        pltpu.SemaphoreType.DMA,
        pltpu.SemaphoreType.DMA,
        pltpu.SemaphoreType.DMA,
        pltpu.SemaphoreType.DMA,
        pltpu.SemaphoreType.DMA,
        pltpu.SemaphoreType.DMA,
        pltpu.VMEM((_STRIP,), jnp.float32),
        pltpu.VMEM((16 * _BLOCKS_PER_STRIP,), jnp.float32),
    ],
    compiler_params=pltpu.CompilerParams(needs_layout_passes=False),
)
def _wf_kernel(pred_hbm, inp_hbm, out_hbm,
               pred_v0, inp_v0, out_v0, pred_v1, inp_v1, out_v1,
               sem_p0, sem_i0, sem_o0, sem_p1, sem_i1, sem_o1,
               p_buf, th_buf):
    nc = 2
    wid = lax.axis_index("s") * nc + lax.axis_index("c")
    base_off = wid * _STRIPS_PER_W * _STRIP
    io = lax.iota(jnp.int32, 16)
    row = io >> 3
    col = io & 7
    base = [(2 * k + row) * 128 + col for k in range(4)]
    kf = (io + 1).astype(jnp.float32)
    inv_k = [1.0 / (kf + (16.0 * j)) for j in range(4)]
    lane15 = jnp.full((16,), 15, jnp.int32)

    slots = (
        (pred_v0, inp_v0, out_v0, sem_p0, sem_i0, sem_o0),
        (pred_v1, inp_v1, out_v1, sem_p1, sem_i1, sem_o1),
    )

    def start_in(s, slot):
        pred_v, inp_v, _, sem_p, sem_i, _ = slot
        off = base_off + s * _STRIP
        pltpu.async_copy(pred_hbm.at[pl.ds(off, _STRIP)], pred_v, sem_p)
        pltpu.async_copy(inp_hbm.at[pl.ds(off, _STRIP)], inp_v, sem_i)

    def wait_in(s, slot):
        pred_v, inp_v, _, sem_p, sem_i, _ = slot
        off = base_off + s * _STRIP
        pltpu.make_async_copy(pred_hbm.at[pl.ds(off, _STRIP)], pred_v, sem_p).wait()
        pltpu.make_async_copy(inp_hbm.at[pl.ds(off, _STRIP)], inp_v, sem_i).wait()

    def wait_out(s, slot):
        _, _, out_v, _, _, sem_o = slot
        off = base_off + s * _STRIP
        pltpu.make_async_copy(out_v, out_hbm.at[pl.ds(off, _STRIP)], sem_o).wait()

    def compute(slot):
        pred_v, inp_v, out_v = slot[0], slot[1], slot[2]

        @plsc.parallel_loop(0, _BLOCKS_PER_STRIP, 1, unroll=3)
        def blk(bi):
            off_b = ((bi >> 4) << 10) + ((bi & 15) << 3)
            idx = [base[k] + off_b for k in range(4)]
            pv = [_expm1_norm(plsc.load_gather(pred_v, [idx[k]])) for k in range(4)]
            for k in range(4):
                p_buf[pl.ds(bi * 64 + k * 16, 16)] = pv[k]
            iv = [_expm1_norm(plsc.load_gather(inp_v, [idx[k]])) for k in range(4)]
            S = _bcast_lane(plsc.cumsum((iv[0] + iv[1]) + (iv[2] + iv[3])),
                            lane15)
            m0, m1, m2, m3 = _sort64(pv[0], pv[1], pv[2], pv[3])
            u = [jnp.flip(m3), jnp.flip(m2), jnp.flip(m1), jnp.flip(m0)]
            acc_m = jnp.zeros((16,), jnp.float32)
            rho_i = jnp.zeros((16,), jnp.int32)
            carry_v = jnp.zeros((16,), jnp.float32)
            for j in range(4):
                cj = plsc.cumsum(u[j]) + carry_v
                carry_v = _bcast_lane(cj, lane15)
                t = (cj - S) * inv_k[j]
                mask = u[j] > t
                acc_m = acc_m + jnp.where(mask, u[j], 0.0)
                rho_i = rho_i + plsc.all_reduce_population_count(mask)
            msum = _bcast_lane(plsc.cumsum(acc_m), lane15)
            rho = rho_i.astype(jnp.float32)
            maxp = _bcast_lane(m3, lane15)
            theta = jnp.where(rho > 0.5, (msum - S) / jnp.maximum(rho, 1.0),
                              maxp - S)
            th_buf[pl.ds(bi * 16, 16)] = theta

        @plsc.parallel_loop(0, _BLOCKS_PER_STRIP, 1, unroll=4)
        def blk_out(bi):
            off_b = ((bi >> 4) << 10) + ((bi & 15) << 3)
            theta = th_buf[pl.ds(bi * 16, 16)]
            for k in range(4):
                pv = p_buf[pl.ds(bi * 64 + k * 16, 16)]
                q = jnp.maximum(pv - theta, 0.0)
                plsc.store_scatter(out_v, [base[k] + off_b], _log1p_over_norm(q))

    def start_out(s, slot):
        _, _, out_v, _, _, sem_o = slot
        off = base_off + s * _STRIP
        pltpu.async_copy(out_v, out_hbm.at[pl.ds(off, _STRIP)], sem_o)

    start_in(0, slots[0])

    def pair_body(ii, carry):
        s0 = ii * 2
        start_in(s0 + 1, slots[1])
        wait_in(s0, slots[0])

        @pl.when(ii > 0)
        def _():
            wait_out(s0 - 2, slots[0])

        compute(slots[0])
        start_out(s0, slots[0])

        @pl.when(ii < _STRIPS_PER_W // 2 - 1)
        def _():
            start_in(s0 + 2, slots[0])

        wait_in(s0 + 1, slots[1])

        @pl.when(ii > 0)
        def _():
            wait_out(s0 - 1, slots[1])

        compute(slots[1])
        start_out(s0 + 1, slots[1])
        return carry

    lax.fori_loop(0, _STRIPS_PER_W // 2, pair_body, 0, unroll=False)
    wait_out(_STRIPS_PER_W - 2, slots[0])
    wait_out(_STRIPS_PER_W - 1, slots[1])


def _to_tile_order(x):
    return (x.reshape(32, 64, 8, 4, 128)
             .transpose(0, 1, 3, 2, 4)
             .reshape(-1))


def kernel(pred_log_norm, input_mosaic_log_norm):
    shape = pred_log_norm.shape
    pf = _to_tile_order(pred_log_norm)
    nf = _to_tile_order(input_mosaic_log_norm)
    out = _wf_kernel(pf, nf)
    return (out.reshape(32, 64, 4, 8, 128)
               .transpose(0, 1, 3, 2, 4)
               .reshape(shape))


# speedup vs baseline: 1.1325x; 1.0022x over previous
import functools

import jax
import jax.numpy as jnp
from jax import lax
from jax.experimental import pallas as pl
from jax.experimental.pallas import tpu as pltpu
from jax.experimental.pallas import tpu_sc as plsc

_NORM = 11.0
_INV_NORM = 1.0 / 11.0
_LN2 = 0.6931471805599453

_W = 512
_STRIP = 8 * _W
_NW = 32
_TOTAL = 2 * 1 * 16 * 512 * 512
_N_STRIPS = _TOTAL // _STRIP
_STRIPS_PER_W = _N_STRIPS // _NW
_BLOCKS_PER_STRIP = _W // 8


def _merge16(a, b):
    rb = jnp.flip(b)
    lo = jnp.minimum(a, rb)
    hi = jnp.maximum(a, rb)
    return jnp.sort(lo), jnp.sort(hi)


def _sort64(v0, v1, v2, v3):
    s0, s1, s2, s3 = jnp.sort(v0), jnp.sort(v1), jnp.sort(v2), jnp.sort(v3)
    a0, a1 = _merge16(s0, s1)
    b0, b1 = _merge16(s2, s3)
    rb0, rb1 = jnp.flip(b1), jnp.flip(b0)
    l0 = jnp.minimum(a0, rb0)
    l1 = jnp.minimum(a1, rb1)
    h0 = jnp.maximum(a0, rb0)
    h1 = jnp.maximum(a1, rb1)
    m0 = jnp.minimum(l0, l1)
    m1 = jnp.maximum(l0, l1)
    m2 = jnp.minimum(h0, h1)
    m3 = jnp.maximum(h0, h1)
    return jnp.sort(m0), jnp.sort(m1), jnp.sort(m2), jnp.sort(m3)


def _log1p_over_norm(q):
    y = q + 1.0
    b = lax.bitcast_convert_type(y, jnp.int32)
    e = (b >> 23) - 127
    m = lax.bitcast_convert_type((b & 0x7FFFFF) | 0x3F800000, jnp.float32)
    ef = e.astype(jnp.float32)
    z = (m - 1.0) / (m + 1.0)
    z2 = z * z
    p = (1.0 / 7.0 * z2 + 0.2) * z2 + 1.0 / 3.0
    p = p * z2 + 1.0
    return (ef * _LN2 + 2.0 * z * p) * _INV_NORM


def _expm1_norm(v):
    return jnp.exp(v * _NORM) - 1.0


def _bcast_lane(x, idx):
    return lax.gather(
        x, idx[:, None],
        dimension_numbers=lax.GatherDimensionNumbers(
            offset_dims=(), collapsed_slice_dims=(0,), start_index_map=(0,)),
        slice_sizes=(1,),
        mode=lax.GatherScatterMode.PROMISE_IN_BOUNDS)


@functools.partial(
    pl.kernel,
    out_type=jax.ShapeDtypeStruct((_TOTAL,), jnp.float32),
    mesh=plsc.VectorSubcoreMesh(core_axis_name="c", subcore_axis_name="s"),
    scratch_types=[
        pltpu.VMEM((_STRIP,), jnp.float32),
        pltpu.VMEM((_STRIP,), jnp.float32),
        pltpu.VMEM((_STRIP,), jnp.float32),
        pltpu.VMEM((_STRIP,), jnp.float32),
        pltpu.VMEM((_STRIP,), jnp.float32),
        pltpu.VMEM((_STRIP,), jnp.float32),
        pltpu.SemaphoreType.DMA,
        pltpu.SemaphoreType.DMA,
        pltpu.SemaphoreType.DMA,
        pltpu.SemaphoreType.DMA,
        pltpu.SemaphoreType.DMA,
        pltpu.SemaphoreType.DMA,
        pltpu.VMEM((_STRIP,), jnp.float32),
        pltpu.VMEM((16 * _BLOCKS_PER_STRIP,), jnp.float32),
    ],
    compiler_params=pltpu.CompilerParams(needs_layout_passes=False),
)
def _wf_kernel(pred_hbm, inp_hbm, out_hbm,
               pred_v0, inp_v0, out_v0, pred_v1, inp_v1, out_v1,
               sem_p0, sem_i0, sem_o0, sem_p1, sem_i1, sem_o1,
               p_buf, th_buf):
    nc = 2
    wid = lax.axis_index("s") * nc + lax.axis_index("c")
    base_off = wid * _STRIPS_PER_W * _STRIP
    io = lax.iota(jnp.int32, 16)
    row = io >> 3
    col = io & 7
    base = [(2 * k + row) * 128 + col for k in range(4)]
    kf = (io + 1).astype(jnp.float32)
    inv_k = [1.0 / (kf + (16.0 * j)) for j in range(4)]
    lane15 = jnp.full((16,), 15, jnp.int32)

    slots = (
        (pred_v0, inp_v0, out_v0, sem_p0, sem_i0, sem_o0),
        (pred_v1, inp_v1, out_v1, sem_p1, sem_i1, sem_o1),
    )

    def start_in(s, slot):
        pred_v, inp_v, _, sem_p, sem_i, _ = slot
        off = base_off + s * _STRIP
        pltpu.async_copy(pred_hbm.at[pl.ds(off, _STRIP)], pred_v, sem_p)
        pltpu.async_copy(inp_hbm.at[pl.ds(off, _STRIP)], inp_v, sem_i)

    def wait_in(s, slot):
        pred_v, inp_v, _, sem_p, sem_i, _ = slot
        off = base_off + s * _STRIP
        pltpu.make_async_copy(pred_hbm.at[pl.ds(off, _STRIP)], pred_v, sem_p).wait()
        pltpu.make_async_copy(inp_hbm.at[pl.ds(off, _STRIP)], inp_v, sem_i).wait()

    def wait_out(s, slot):
        _, _, out_v, _, _, sem_o = slot
        off = base_off + s * _STRIP
        pltpu.make_async_copy(out_v, out_hbm.at[pl.ds(off, _STRIP)], sem_o).wait()

    def compute(slot):
        pred_v, inp_v, out_v = slot[0], slot[1], slot[2]

        @plsc.parallel_loop(0, _BLOCKS_PER_STRIP, 1, unroll=3)
        def blk(bi):
            off_b = ((bi >> 4) << 10) + ((bi & 15) << 3)
            idx = [base[k] + off_b for k in range(4)]
            pv = [_expm1_norm(plsc.load_gather(pred_v, [idx[k]])) for k in range(4)]
            for k in range(4):
                p_buf[pl.ds(bi * 64 + k * 16, 16)] = pv[k]
            iv = [_expm1_norm(plsc.load_gather(inp_v, [idx[k]])) for k in range(4)]
            S = _bcast_lane(plsc.cumsum((iv[0] + iv[1]) + (iv[2] + iv[3])),
                            lane15)
            m0, m1, m2, m3 = _sort64(pv[0], pv[1], pv[2], pv[3])
            u = [jnp.flip(m3), jnp.flip(m2), jnp.flip(m1), jnp.flip(m0)]
            acc_m = jnp.zeros((16,), jnp.float32)
            rho_i = jnp.zeros((16,), jnp.int32)
            carry_v = jnp.zeros((16,), jnp.float32)
            for j in range(4):
                cj = plsc.cumsum(u[j]) + carry_v
                carry_v = _bcast_lane(cj, lane15)
                t = (cj - S) * inv_k[j]
                mask = u[j] > t
                acc_m = acc_m + jnp.where(mask, u[j], 0.0)
                rho_i = rho_i + plsc.all_reduce_population_count(mask)
            msum = _bcast_lane(plsc.cumsum(acc_m), lane15)
            rho = rho_i.astype(jnp.float32)
            maxp = _bcast_lane(m3, lane15)
            theta = jnp.where(rho > 0.5, (msum - S) / jnp.maximum(rho, 1.0),
                              maxp - S)
            th_buf[pl.ds(bi * 16, 16)] = theta

        @plsc.parallel_loop(0, _BLOCKS_PER_STRIP, 1, unroll=4)
        def blk_out(bi):
            off_b = ((bi >> 4) << 10) + ((bi & 15) << 3)
            theta = th_buf[pl.ds(bi * 16, 16)]
            for k in range(4):
                pv = p_buf[pl.ds(bi * 64 + k * 16, 16)]
                q = jnp.maximum(pv - theta, 0.0)
                plsc.store_scatter(out_v, [base[k] + off_b], _log1p_over_norm(q))

    def start_out(s, slot):
        _, _, out_v, _, _, sem_o = slot
        off = base_off + s * _STRIP
        pltpu.async_copy(out_v, out_hbm.at[pl.ds(off, _STRIP)], sem_o)

    start_in(0, slots[0])

    def pair_body(ii, carry):
        s0 = ii * 2
        start_in(s0 + 1, slots[1])
        wait_in(s0, slots[0])

        @pl.when(ii > 0)
        def _():
            wait_out(s0 - 2, slots[0])

        compute(slots[0])
        start_out(s0, slots[0])

        @pl.when(ii < _STRIPS_PER_W // 2 - 1)
        def _():
            start_in(s0 + 2, slots[0])

        wait_in(s0 + 1, slots[1])

        @pl.when(ii > 0)
        def _():
            wait_out(s0 - 1, slots[1])

        compute(slots[1])
        start_out(s0 + 1, slots[1])
        return carry

    lax.fori_loop(0, _STRIPS_PER_W // 2, pair_body, 0, unroll=False)
    wait_out(_STRIPS_PER_W - 2, slots[0])
    wait_out(_STRIPS_PER_W - 1, slots[1])


def _to_tile_order(x):
    return (x.reshape(32, 64, 8, 4, 128)
             .transpose(0, 1, 3, 2, 4)
             .reshape(-1))


def kernel(pred_log_norm, input_mosaic_log_norm):
    shape = pred_log_norm.shape
    pf = _to_tile_order(pred_log_norm)
    nf = _to_tile_order(input_mosaic_log_norm)
    out = _wf_kernel(pf, nf)
    return (out.reshape(32, 64, 4, 8, 128)
               .transpose(0, 1, 3, 2, 4)
               .reshape(shape))
